# Initial kernel scaffold; baseline (speedup 1.0000x reference)
import jax
import jax.numpy as jnp
from jax.experimental import pallas as pl


def kernel(x):
    raise NotImplementedError("write your pallas kernel here")


# speedup vs baseline: 40.8475x; 40.8475x over previous
import functools

import jax
import jax.numpy as jnp
from jax.experimental import pallas as pl

N = 32
NT = 465
SCALE = float(N + 1)
BATCH_SUB = 8


def _offset(c):
    return 30 * c - c * (c - 1) // 2


def _fidx(r, c):
    return _offset(c) + (r - (c + 2))


def _fmatrix_kernel(x_ref, out_ref, loss_ref):
    out_ref[...] = x_ref[...] * SCALE

    def col(k):
        return out_ref[k]

    acc = jnp.zeros((BATCH_SUB, 128), jnp.float32)
    for i in range(2, N):
        prev = jnp.float32(1.0) if i == 2 else col(_fidx(i - 1, 0))
        curr = col(_fidx(i, 0))
        lower = jnp.maximum(prev - 1.0, 0.0)
        new = jnp.maximum(jnp.minimum(curr, prev), lower)
        acc = acc + jnp.abs(new - curr)
        out_ref[_fidx(i, 0)] = new
        for j in range(1, i - 1):
            up = jnp.float32(i - 1.0) if j == i - 2 else col(_fidx(i - 1, j))
            left = col(_fidx(i, j - 1))
            diag = col(_fidx(i - 1, j - 1))
            curr = col(_fidx(i, j))
            t = left - diag
            l2 = up - 1.0
            lower = jnp.maximum(jnp.maximum(left, l2), jnp.maximum(t + l2, 0.0))
            u_raw = jnp.minimum(up, t + up)
            new = jnp.maximum(jnp.minimum(curr, u_raw), lower)
            acc = acc + jnp.abs(new - curr)
            out_ref[_fidx(i, j)] = new
    out_ref[...] = out_ref[...] * (1.0 / SCALE)
    loss_ref[0] = acc


@functools.partial(jax.jit, static_argnames=())
def _run(x):
    b = x.shape[0]
    lanes = b // 128
    grid = lanes // BATCH_SUB
    xt = jnp.transpose(x).reshape(NT, lanes, 128)
    out_t, loss_parts = pl.pallas_call(
        _fmatrix_kernel,
        grid=(grid,),
        in_specs=[
            pl.BlockSpec((NT, BATCH_SUB, 128), lambda g: (0, g, 0)),
        ],
        out_specs=[
            pl.BlockSpec((NT, BATCH_SUB, 128), lambda g: (0, g, 0)),
            pl.BlockSpec((1, BATCH_SUB, 128), lambda g: (g, 0, 0)),
        ],
        out_shape=[
            jax.ShapeDtypeStruct((NT, lanes, 128), jnp.float32),
            jax.ShapeDtypeStruct((grid, BATCH_SUB, 128), jnp.float32),
        ],
    )(xt)
    out = jnp.transpose(out_t.reshape(NT, b))
    return out, jnp.sum(loss_parts)


def kernel(x):
    return _run(x)
